# baseline (speedup 1.0000x reference)
import functools

import jax
import jax.numpy as jnp
from jax import lax
from jax.experimental import pallas as pl
from jax.experimental.pallas import tpu as pltpu
from jax.experimental.pallas import tpu_sc as plsc

_B = 16384
_D = 32
_NC = 2
_NS = 16
_NW = _NC * _NS
_BPW = _B // _NW


def _body(idx_hbm, sp_hbm, su_hbm, out_sp, out_su, idx_v, sem):
    wid = lax.axis_index("s") * _NC + lax.axis_index("c")
    base = wid * _BPW
    pltpu.sync_copy(idx_hbm.at[pl.ds(base, _BPW)], idx_v)

    def issue(g, carry):
        vec = idx_v[pl.ds(g * 16, 16)]
        for l in range(16):
            r = vec[l]
            j = base + g * 16 + l
            pltpu.async_copy(sp_hbm.at[pl.ds(r, 1)], out_sp.at[pl.ds(j, 1)], sem)
            pltpu.async_copy(su_hbm.at[pl.ds(r, 1)], out_su.at[pl.ds(j, 1)], sem)
        return carry

    lax.fori_loop(0, _BPW // 16, issue, 0)

    def drain(j, carry):
        pltpu.make_async_copy(sp_hbm.at[pl.ds(0, 1)], out_sp.at[pl.ds(base, 1)], sem).wait()
        pltpu.make_async_copy(su_hbm.at[pl.ds(0, 1)], out_su.at[pl.ds(base, 1)], sem).wait()
        return carry

    lax.fori_loop(0, _BPW, drain, 0)


@jax.jit
def kernel(node_indices, B_sp, B_su):
    gather = pl.kernel(
        _body,
        out_type=(
            jax.ShapeDtypeStruct((_B, _D), jnp.float32),
            jax.ShapeDtypeStruct((_B, _D), jnp.float32),
        ),
        mesh=plsc.VectorSubcoreMesh(core_axis_name="c", subcore_axis_name="s"),
        scratch_types=[
            pltpu.VMEM((_BPW,), jnp.int32),
            pltpu.SemaphoreType.DMA,
        ],
        compiler_params=pltpu.CompilerParams(use_tc_tiling_on_sc=True),
    )
    return gather(node_indices.astype(jnp.int32), B_sp, B_su)
